# baseline (speedup 1.0000x reference)
import functools
import math

import jax
import jax.numpy as jnp
from jax import lax
from jax.experimental import pallas as pl
from jax.experimental.pallas import tpu as pltpu
from jax.experimental.pallas import tpu_sc as plsc

D_MODEL = 64
SCALE = float(math.sqrt(D_MODEL))
VOCAB = 1000000
NW = 32
L = 16

A_CHUNK = 128
A_UNITS = VOCAB // A_CHUNK
A_TAIL = VOCAB - A_UNITS * A_CHUNK
A_NBUF = 3


def _table_kernel(wt_hbm, wtail_hbm, tab_hbm, ibuf, obuf, tbuf, *sems):
    isems = sems[:A_NBUF]
    osems = sems[A_NBUF:]
    wid = lax.axis_index("s") * 2 + lax.axis_index("c")
    n_units = A_UNITS // NW + jnp.where(wid < A_UNITS % NW, 1, 0)

    def start_in(s, c):
        pltpu.make_async_copy(
            wt_hbm.at[:, pl.ds(c * A_CHUNK, A_CHUNK)], ibuf.at[s],
            isems[s]).start()

    def wait_in(s):
        pltpu.make_async_copy(
            wt_hbm.at[:, pl.ds(0, A_CHUNK)], ibuf.at[s], isems[s]).wait()

    def start_out(s, c):
        pltpu.make_async_copy(
            obuf.at[s], tab_hbm.at[pl.ds(c * (A_CHUNK // 2), A_CHUNK // 2)],
            osems[s]).start()

    def wait_out(s, c):
        pltpu.make_async_copy(
            obuf.at[s], tab_hbm.at[pl.ds(c * (A_CHUNK // 2), A_CHUNK // 2)],
            osems[s]).wait()

    rowv = [lax.broadcasted_iota(jnp.int32, (L,), 0) + L * k for k in range(D_MODEL // L)]

    def transpose_unit(s):
        def body(q, carry):
            for half in range(2):
                col = 2 * q + half
                for k in range(D_MODEL // L):
                    v = plsc.load_gather(
                        ibuf.at[s], [rowv[k], jnp.full((L,), col,
                                                       jnp.int32)])
                    obuf[s, q, pl.ds(half * D_MODEL + k * L, L)] = v
            return carry
        lax.fori_loop(0, A_CHUNK // 2, body, 0, unroll=2)

    def unit_for(i):
        return i * NW + wid

    for s in range(A_NBUF):
        start_in(s, unit_for(s))

    n_rounds = (A_UNITS // NW + 1 + A_NBUF - 1) // A_NBUF

    def round_body(r, carry):
        for s in range(A_NBUF):
            i = r * A_NBUF + s
            @pl.when(i < n_units)
            def _():
                wait_in(s)
                @pl.when(i >= A_NBUF)
                def _():
                    wait_out(s, 0)
                transpose_unit(s)
                start_out(s, unit_for(i))
                @pl.when(i + A_NBUF < n_units)
                def _():
                    start_in(s, unit_for(i + A_NBUF))
        return carry
    lax.fori_loop(0, n_rounds, round_body, 0)

    for s in range(A_NBUF):
        wait_out(s, 0)

    @pl.when(wid == 0)
    def _tail():
        pltpu.sync_copy(wtail_hbm, tbuf)
        pltpu.sync_copy(
            tbuf, tab_hbm.at[pl.ds(A_UNITS * (A_CHUNK // 2), A_TAIL // 2)])


N_T = 200
N_B = 4096
B_NBUF = 3


def _lookup_kernel(tab_hbm, xt_hbm, out_hbm, vbuf, gbuf, obuf, gidx, *sems):
    gsems = sems[:B_NBUF]
    osems = sems[B_NBUF:]
    wid = lax.axis_index("s") * 2 + lax.axis_index("c")
    pltpu.sync_copy(xt_hbm.at[:, pl.ds(wid * 128, 128)], vbuf)

    rowv = [lax.broadcasted_iota(jnp.int32, (L,), 0) + L * g for g in range(8)]

    def prep_idx(s, t):
        for g in range(8):
            v = vbuf[t, pl.ds(g * L, L)]
            gidx[s, pl.ds(g * L, L)] = lax.shift_right_logical(v, 1)

    def start_gather(s):
        pltpu.make_async_copy(
            tab_hbm.at[gidx.at[s]], gbuf.at[s], gsems[s]).start()

    def wait_gather(s):
        pltpu.make_async_copy(
            tab_hbm.at[gidx.at[0]], gbuf.at[s], gsems[s]).wait()

    def start_out(s, t):
        pltpu.make_async_copy(
            obuf.at[s], out_hbm.at[t, :, pl.ds(wid * 128, 128)],
            osems[s]).start()

    def wait_out(s, t):
        pltpu.make_async_copy(
            obuf.at[s], out_hbm.at[t, :, pl.ds(wid * 128, 128)],
            osems[s]).wait()

    def transpose_unit(s, t):
        for g in range(8):
            v = vbuf[t, pl.ds(g * L, L)]
            colbase = lax.mul(lax.bitwise_and(v, 1), D_MODEL)

            def body(d, carry):
                val = plsc.load_gather(gbuf.at[s], [rowv[g], colbase + d])
                obuf[s, d, pl.ds(g * L, L)] = val * SCALE
                return carry
            lax.fori_loop(0, D_MODEL, body, 0, unroll=4)

    for s in range(B_NBUF):
        prep_idx(s, s)
        start_gather(s)

    n_rounds = (N_T + B_NBUF - 1) // B_NBUF

    def round_body(r, carry):
        for s in range(B_NBUF):
            t = r * B_NBUF + s
            @pl.when(t < N_T)
            def _():
                wait_gather(s)
                @pl.when(t >= B_NBUF)
                def _():
                    wait_out(s, 0)
                transpose_unit(s, t)
                start_out(s, t)
                @pl.when(t + B_NBUF < N_T)
                def _():
                    prep_idx(s, t + B_NBUF)
                    start_gather(s)
        return carry
    lax.fori_loop(0, n_rounds, round_body, 0)

    for s in range(B_NBUF):
        wait_out(s, 0)


def _build_table():
    mesh = plsc.VectorSubcoreMesh(core_axis_name="c", subcore_axis_name="s")
    return functools.partial(
        pl.kernel,
        mesh=mesh,
        out_type=jax.ShapeDtypeStruct((VOCAB // 2, 128), jnp.float32),
        scratch_types=[
            pltpu.VMEM((A_NBUF, D_MODEL, A_CHUNK), jnp.float32),
            pltpu.VMEM((A_NBUF, A_CHUNK // 2, 128), jnp.float32),
            pltpu.VMEM((A_TAIL // 2, 128), jnp.float32),
        ] + [pltpu.SemaphoreType.DMA] * (2 * A_NBUF),
        compiler_params=pltpu.CompilerParams(use_tc_tiling_on_sc=True, needs_layout_passes=False),
    )(_table_kernel)


def _build_lookup():
    mesh = plsc.VectorSubcoreMesh(core_axis_name="c", subcore_axis_name="s")
    return functools.partial(
        pl.kernel,
        mesh=mesh,
        out_type=jax.ShapeDtypeStruct((N_T, D_MODEL, N_B), jnp.float32),
        scratch_types=[
            pltpu.VMEM((N_T, 128), jnp.int32),
            pltpu.VMEM((B_NBUF, 128, 128), jnp.float32),
            pltpu.VMEM((B_NBUF, D_MODEL, 128), jnp.float32),
            pltpu.VMEM((B_NBUF, 128), jnp.int32),
        ] + [pltpu.SemaphoreType.DMA] * (2 * B_NBUF),
        compiler_params=pltpu.CompilerParams(use_tc_tiling_on_sc=True, needs_layout_passes=False),
    )(_lookup_kernel)


@jax.jit
def kernel(x, W):
    xt = jnp.swapaxes(x.astype(jnp.int32), 0, 1)
    wt = jnp.swapaxes(W, 0, 1)
    wtail = jnp.reshape(W[VOCAB - A_TAIL:, :], (A_TAIL // 2, 128))
    tab = _build_table()(wt, wtail)
    out_t = _build_lookup()(tab, xt)
    return jnp.transpose(out_t, (2, 0, 1))


# speedup vs baseline: 1.9381x; 1.9381x over previous
import functools
import math

import jax
import jax.numpy as jnp
from jax import lax
from jax.experimental import pallas as pl
from jax.experimental.pallas import tpu as pltpu
from jax.experimental.pallas import tpu_sc as plsc

D_MODEL = 64
SCALE = float(math.sqrt(D_MODEL))
VOCAB = 1000000
NW = 32
L = 16

A_CHUNK = 128
A_UNITS = VOCAB // A_CHUNK
A_TAIL = VOCAB - A_UNITS * A_CHUNK
A_NBUF = 3


def _table_kernel(wt_hbm, wtail_hbm, tab_hbm, ibuf, obuf, tbuf, *sems):
    isems = sems[:A_NBUF]
    osems = sems[A_NBUF:]
    wid = lax.axis_index("s") * 2 + lax.axis_index("c")
    n_units = A_UNITS // NW + jnp.where(wid < A_UNITS % NW, 1, 0)

    def start_in(s, c):
        pltpu.make_async_copy(
            wt_hbm.at[:, pl.ds(c * A_CHUNK, A_CHUNK)], ibuf.at[s],
            isems[s]).start()

    def wait_in(s):
        pltpu.make_async_copy(
            wt_hbm.at[:, pl.ds(0, A_CHUNK)], ibuf.at[s], isems[s]).wait()

    def start_out(s, c):
        pltpu.make_async_copy(
            obuf.at[s], tab_hbm.at[pl.ds(c * (A_CHUNK // 2), A_CHUNK // 2)],
            osems[s]).start()

    def wait_out(s, c):
        pltpu.make_async_copy(
            obuf.at[s], tab_hbm.at[pl.ds(c * (A_CHUNK // 2), A_CHUNK // 2)],
            osems[s]).wait()

    rowv = [lax.broadcasted_iota(jnp.int32, (L,), 0) + L * k for k in range(D_MODEL // L)]

    def transpose_unit(s):
        @plsc.parallel_loop(0, A_CHUNK // 2, unroll=4)
        def body(q):
            for half in range(2):
                col = 2 * q + half
                for k in range(D_MODEL // L):
                    v = plsc.load_gather(
                        ibuf.at[s], [rowv[k], jnp.full((L,), col,
                                                       jnp.int32)])
                    obuf[s, q, pl.ds(half * D_MODEL + k * L, L)] = v

    def unit_for(i):
        return i * NW + wid

    for s in range(A_NBUF):
        start_in(s, unit_for(s))

    n_rounds = (A_UNITS // NW + 1 + A_NBUF - 1) // A_NBUF

    def round_body(r, carry):
        for s in range(A_NBUF):
            i = r * A_NBUF + s
            @pl.when(i < n_units)
            def _():
                wait_in(s)
                @pl.when(i >= A_NBUF)
                def _():
                    wait_out(s, 0)
                transpose_unit(s)
                start_out(s, unit_for(i))
                @pl.when(i + A_NBUF < n_units)
                def _():
                    start_in(s, unit_for(i + A_NBUF))
        return carry
    lax.fori_loop(0, n_rounds, round_body, 0)

    for s in range(A_NBUF):
        wait_out(s, 0)

    @pl.when(wid == 0)
    def _tail():
        pltpu.sync_copy(wtail_hbm, tbuf)
        pltpu.sync_copy(
            tbuf, tab_hbm.at[pl.ds(A_UNITS * (A_CHUNK // 2), A_TAIL // 2)])


N_T = 200
N_B = 4096
B_NBUF = 3


def _lookup_kernel(tab_hbm, xt_hbm, out_hbm, vbuf, gbuf, obuf, gidx, *sems):
    gsems = sems[:B_NBUF]
    osems = sems[B_NBUF:]
    wid = lax.axis_index("s") * 2 + lax.axis_index("c")
    pltpu.sync_copy(xt_hbm.at[:, pl.ds(wid * 128, 128)], vbuf)

    rowv = [lax.broadcasted_iota(jnp.int32, (L,), 0) + L * g for g in range(8)]

    def prep_idx(s, t):
        for g in range(8):
            v = vbuf[t, pl.ds(g * L, L)]
            gidx[s, pl.ds(g * L, L)] = lax.shift_right_logical(v, 1)

    def start_gather(s):
        pltpu.make_async_copy(
            tab_hbm.at[gidx.at[s]], gbuf.at[s], gsems[s]).start()

    def wait_gather(s):
        pltpu.make_async_copy(
            tab_hbm.at[gidx.at[0]], gbuf.at[s], gsems[s]).wait()

    def start_out(s, t):
        pltpu.make_async_copy(
            obuf.at[s], out_hbm.at[t, :, pl.ds(wid * 128, 128)],
            osems[s]).start()

    def wait_out(s, t):
        pltpu.make_async_copy(
            obuf.at[s], out_hbm.at[t, :, pl.ds(wid * 128, 128)],
            osems[s]).wait()

    def transpose_unit(s, t):
        for g in range(8):
            v = vbuf[t, pl.ds(g * L, L)]
            colbase = lax.mul(lax.bitwise_and(v, 1), D_MODEL)

            @plsc.parallel_loop(0, D_MODEL, unroll=8)
            def body(d):
                val = plsc.load_gather(gbuf.at[s], [rowv[g], colbase + d])
                obuf[s, d, pl.ds(g * L, L)] = val * SCALE

    for s in range(B_NBUF):
        prep_idx(s, s)
        start_gather(s)

    n_rounds = (N_T + B_NBUF - 1) // B_NBUF

    def round_body(r, carry):
        for s in range(B_NBUF):
            t = r * B_NBUF + s
            @pl.when(t < N_T)
            def _():
                wait_gather(s)
                @pl.when(t >= B_NBUF)
                def _():
                    wait_out(s, 0)
                transpose_unit(s, t)
                start_out(s, t)
                @pl.when(t + B_NBUF < N_T)
                def _():
                    prep_idx(s, t + B_NBUF)
                    start_gather(s)
        return carry
    lax.fori_loop(0, n_rounds, round_body, 0)

    for s in range(B_NBUF):
        wait_out(s, 0)


def _build_table():
    mesh = plsc.VectorSubcoreMesh(core_axis_name="c", subcore_axis_name="s")
    return functools.partial(
        pl.kernel,
        mesh=mesh,
        out_type=jax.ShapeDtypeStruct((VOCAB // 2, 128), jnp.float32),
        scratch_types=[
            pltpu.VMEM((A_NBUF, D_MODEL, A_CHUNK), jnp.float32),
            pltpu.VMEM((A_NBUF, A_CHUNK // 2, 128), jnp.float32),
            pltpu.VMEM((A_TAIL // 2, 128), jnp.float32),
        ] + [pltpu.SemaphoreType.DMA] * (2 * A_NBUF),
        compiler_params=pltpu.CompilerParams(use_tc_tiling_on_sc=True, needs_layout_passes=False),
    )(_table_kernel)


def _build_lookup():
    mesh = plsc.VectorSubcoreMesh(core_axis_name="c", subcore_axis_name="s")
    return functools.partial(
        pl.kernel,
        mesh=mesh,
        out_type=jax.ShapeDtypeStruct((N_T, D_MODEL, N_B), jnp.float32),
        scratch_types=[
            pltpu.VMEM((N_T, 128), jnp.int32),
            pltpu.VMEM((B_NBUF, 128, 128), jnp.float32),
            pltpu.VMEM((B_NBUF, D_MODEL, 128), jnp.float32),
            pltpu.VMEM((B_NBUF, 128), jnp.int32),
        ] + [pltpu.SemaphoreType.DMA] * (2 * B_NBUF),
        compiler_params=pltpu.CompilerParams(use_tc_tiling_on_sc=True, needs_layout_passes=False),
    )(_lookup_kernel)


@jax.jit
def kernel(x, W):
    xt = jnp.swapaxes(x.astype(jnp.int32), 0, 1)
    wt = jnp.swapaxes(W, 0, 1)
    wtail = jnp.reshape(W[VOCAB - A_TAIL:, :], (A_TAIL // 2, 128))
    tab = _build_table()(wt, wtail)
    out_t = _build_lookup()(tab, xt)
    return jnp.transpose(out_t, (2, 0, 1))


# speedup vs baseline: 3.1922x; 1.6471x over previous
import functools
import math

import jax
import jax.numpy as jnp
from jax import lax
from jax.experimental import pallas as pl
from jax.experimental.pallas import tpu as pltpu
from jax.experimental.pallas import tpu_sc as plsc

D_MODEL = 64
SCALE = float(math.sqrt(D_MODEL))
VOCAB = 1000000
NW = 32
L = 16

A_CHUNK = 128
A_UNITS = VOCAB // A_CHUNK
A_TAIL = VOCAB - A_UNITS * A_CHUNK
A_NBUF = 3


def _table_kernel(wt_hbm, wtail_hbm, tab_hbm, ibuf, obuf, tbuf, *sems):
    isems = sems[:A_NBUF]
    osems = sems[A_NBUF:]
    wid = lax.axis_index("s") * 2 + lax.axis_index("c")
    n_units = A_UNITS // NW + jnp.where(wid < A_UNITS % NW, 1, 0)

    def start_in(s, c):
        pltpu.make_async_copy(
            wt_hbm.at[:, pl.ds(c * A_CHUNK, A_CHUNK)], ibuf.at[s],
            isems[s]).start()

    def wait_in(s):
        pltpu.make_async_copy(
            wt_hbm.at[:, pl.ds(0, A_CHUNK)], ibuf.at[s], isems[s]).wait()

    def start_out(s, c):
        pltpu.make_async_copy(
            obuf.at[s], tab_hbm.at[pl.ds(c * A_CHUNK, A_CHUNK)],
            osems[s]).start()

    def wait_out(s):
        pltpu.make_async_copy(
            obuf.at[s], tab_hbm.at[pl.ds(0, A_CHUNK)], osems[s]).wait()

    iota = lax.broadcasted_iota(jnp.int32, (L,), 0)

    def transpose_unit(s):
        for k in range(D_MODEL // L):
            for cb in range(A_CHUNK // L):
                cvec = iota + cb * L

                @plsc.parallel_loop(0, L, unroll=4)
                def body(j):
                    dvec = lax.rem(iota + j, L) + k * L
                    val = plsc.load_gather(ibuf.at[s], [dvec, cvec])
                    plsc.store_scatter(obuf.at[s], [cvec, dvec], val)

    for s in range(A_NBUF):
        start_in(s, s * NW + wid)

    n_rounds = (A_UNITS // NW + 1 + A_NBUF - 1) // A_NBUF

    def round_body(r, carry):
        for s in range(A_NBUF):
            i = r * A_NBUF + s

            @pl.when(i < n_units)
            def _():
                wait_in(s)

                @pl.when(i >= A_NBUF)
                def _():
                    wait_out(s)
                transpose_unit(s)
                start_out(s, i * NW + wid)

                @pl.when(i + A_NBUF < n_units)
                def _():
                    start_in(s, (i + A_NBUF) * NW + wid)
        return carry
    lax.fori_loop(0, n_rounds, round_body, 0)

    for s in range(A_NBUF):
        wait_out(s)

    @pl.when(wid == 0)
    def _tail():
        pltpu.sync_copy(wtail_hbm, tbuf)
        pltpu.sync_copy(tbuf, tab_hbm.at[pl.ds(A_UNITS * A_CHUNK, A_TAIL)])


B_CHUNK = 128
N_LOOK = 4096 * 200
B_UNITS = N_LOOK // (NW * B_CHUNK)
B_NBUF = 3


def _lookup_kernel(tab_hbm, idx_hbm, out_hbm, idx_v, gbuf, obuf, *sems):
    gsems = sems[:B_NBUF]
    osems = sems[B_NBUF:]
    wid = lax.axis_index("s") * 2 + lax.axis_index("c")
    base = wid * (B_UNITS * B_CHUNK)
    pltpu.sync_copy(idx_hbm.at[wid], idx_v)

    def start_gather(s, u):
        pltpu.make_async_copy(
            tab_hbm.at[idx_v.at[u]], gbuf.at[s], gsems[s]).start()

    def wait_gather(s):
        pltpu.make_async_copy(
            tab_hbm.at[idx_v.at[0]], gbuf.at[s], gsems[s]).wait()

    def start_out(s, u):
        pltpu.make_async_copy(
            obuf.at[s], out_hbm.at[pl.ds(base + u * B_CHUNK, B_CHUNK)],
            osems[s]).start()

    def wait_out(s):
        pltpu.make_async_copy(
            obuf.at[s], out_hbm.at[pl.ds(0, B_CHUNK)], osems[s]).wait()

    def scale_unit(s):
        @plsc.parallel_loop(0, B_CHUNK, unroll=4)
        def body(r):
            for k in range(D_MODEL // L):
                obuf[s, r, pl.ds(k * L, L)] = (
                    gbuf[s, r, pl.ds(k * L, L)] * SCALE)

    for s in range(B_NBUF):
        start_gather(s, s)

    n_rounds = (B_UNITS + B_NBUF - 1) // B_NBUF

    def round_body(r, carry):
        for s in range(B_NBUF):
            u = r * B_NBUF + s

            @pl.when(u < B_UNITS)
            def _():
                wait_gather(s)

                @pl.when(u >= B_NBUF)
                def _():
                    wait_out(s)
                scale_unit(s)
                start_out(s, u)

                @pl.when(u + B_NBUF < B_UNITS)
                def _():
                    start_gather(s, u + B_NBUF)
        return carry
    lax.fori_loop(0, n_rounds, round_body, 0)

    for s in range(B_NBUF):
        wait_out(s)


def _build_table():
    mesh = plsc.VectorSubcoreMesh(core_axis_name="c", subcore_axis_name="s")
    return functools.partial(
        pl.kernel,
        mesh=mesh,
        out_type=jax.ShapeDtypeStruct((VOCAB, 128), jnp.float32),
        scratch_types=[
            pltpu.VMEM((A_NBUF, D_MODEL, A_CHUNK), jnp.float32),
            pltpu.VMEM((A_NBUF, A_CHUNK, 128), jnp.float32),
            pltpu.VMEM((A_TAIL, 128), jnp.float32),
        ] + [pltpu.SemaphoreType.DMA] * (2 * A_NBUF),
        compiler_params=pltpu.CompilerParams(
            use_tc_tiling_on_sc=True, needs_layout_passes=False),
    )(_table_kernel)


def _build_lookup():
    mesh = plsc.VectorSubcoreMesh(core_axis_name="c", subcore_axis_name="s")
    return functools.partial(
        pl.kernel,
        mesh=mesh,
        out_type=jax.ShapeDtypeStruct((N_LOOK, D_MODEL), jnp.float32),
        scratch_types=[
            pltpu.VMEM((B_UNITS, B_CHUNK), jnp.int32),
            pltpu.VMEM((B_NBUF, B_CHUNK, 128), jnp.float32),
            pltpu.VMEM((B_NBUF, B_CHUNK, D_MODEL), jnp.float32),
        ] + [pltpu.SemaphoreType.DMA] * (2 * B_NBUF),
        compiler_params=pltpu.CompilerParams(
            use_tc_tiling_on_sc=True, needs_layout_passes=False),
    )(_lookup_kernel)


@jax.jit
def kernel(x, W):
    wt = jnp.swapaxes(W, 0, 1)
    wtail = jnp.pad(W[VOCAB - A_TAIL:, :], ((0, 0), (0, 128 - D_MODEL)))
    tab = _build_table()(wt, wtail)
    idx = x.astype(jnp.int32).reshape(NW, B_UNITS, B_CHUNK)
    out = _build_lookup()(tab, idx)
    return out.reshape(4096, 200, D_MODEL)


# speedup vs baseline: 3.7397x; 1.1715x over previous
import functools
import math

import jax
import jax.numpy as jnp
from jax import lax
from jax.experimental import pallas as pl
from jax.experimental.pallas import tpu as pltpu
from jax.experimental.pallas import tpu_sc as plsc

D_MODEL = 64
SCALE = float(math.sqrt(D_MODEL))
VOCAB = 1000000
NW = 32
L = 16

A_CHUNK = 128
A_UNITS = VOCAB // A_CHUNK
A_TAIL = VOCAB - A_UNITS * A_CHUNK
A_NBUF = 3


def _table_kernel(wt_hbm, wtail_hbm, tab_hbm, ibuf, obuf, tbuf, *sems):
    isems = sems[:A_NBUF]
    osems = sems[A_NBUF:]
    wid = lax.axis_index("s") * 2 + lax.axis_index("c")
    n_units = A_UNITS // NW + jnp.where(wid < A_UNITS % NW, 1, 0)

    def start_in(s, c):
        pltpu.make_async_copy(
            wt_hbm.at[:, pl.ds(c * A_CHUNK, A_CHUNK)], ibuf.at[s],
            isems[s]).start()

    def wait_in(s):
        pltpu.make_async_copy(
            wt_hbm.at[:, pl.ds(0, A_CHUNK)], ibuf.at[s], isems[s]).wait()

    def start_out(s, c):
        pltpu.make_async_copy(
            obuf.at[s], tab_hbm.at[pl.ds(c * A_CHUNK, A_CHUNK)],
            osems[s]).start()

    def wait_out(s):
        pltpu.make_async_copy(
            obuf.at[s], tab_hbm.at[pl.ds(0, A_CHUNK)], osems[s]).wait()

    iota = lax.broadcasted_iota(jnp.int32, (L,), 0)

    def transpose_unit(s):
        @plsc.parallel_loop(0, L, unroll=2)
        def body(j):
            rotv = lax.rem(iota + j, L)
            for k in range(D_MODEL // L):
                dvec = rotv + k * L
                for cb in range(A_CHUNK // L):
                    cvec = iota + cb * L
                    val = plsc.load_gather(ibuf.at[s], [dvec, cvec])
                    plsc.store_scatter(obuf.at[s], [cvec, dvec], val)

    for s in range(A_NBUF):
        start_in(s, s * NW + wid)

    n_rounds = (A_UNITS // NW + 1 + A_NBUF - 1) // A_NBUF

    def round_body(r, carry):
        for s in range(A_NBUF):
            i = r * A_NBUF + s

            @pl.when(i < n_units)
            def _():
                wait_in(s)

                @pl.when(i >= A_NBUF)
                def _():
                    wait_out(s)
                transpose_unit(s)
                start_out(s, i * NW + wid)

                @pl.when(i + A_NBUF < n_units)
                def _():
                    start_in(s, (i + A_NBUF) * NW + wid)
        return carry
    lax.fori_loop(0, n_rounds, round_body, 0)

    for s in range(A_NBUF):
        wait_out(s)

    @pl.when(wid == 0)
    def _tail():
        pltpu.sync_copy(wtail_hbm, tbuf)
        pltpu.sync_copy(tbuf, tab_hbm.at[pl.ds(A_UNITS * A_CHUNK, A_TAIL)])


B_CHUNK = 128
N_LOOK = 4096 * 200
B_UNITS = N_LOOK // (NW * B_CHUNK)
B_NBUF = 3


def _lookup_kernel(tab_hbm, idx_hbm, out_hbm, idx_v, gbuf, obuf, *sems):
    gsems = sems[:B_NBUF]
    osems = sems[B_NBUF:]
    wid = lax.axis_index("s") * 2 + lax.axis_index("c")
    base = wid * (B_UNITS * B_CHUNK)
    pltpu.sync_copy(idx_hbm.at[wid], idx_v)

    def start_gather(s, u):
        pltpu.make_async_copy(
            tab_hbm.at[idx_v.at[u]], gbuf.at[s], gsems[s]).start()

    def wait_gather(s):
        pltpu.make_async_copy(
            tab_hbm.at[idx_v.at[0]], gbuf.at[s], gsems[s]).wait()

    def start_out(s, u):
        pltpu.make_async_copy(
            obuf.at[s], out_hbm.at[pl.ds(base + u * B_CHUNK, B_CHUNK)],
            osems[s]).start()

    def wait_out(s):
        pltpu.make_async_copy(
            obuf.at[s], out_hbm.at[pl.ds(0, B_CHUNK)], osems[s]).wait()

    def scale_unit(s):
        @plsc.parallel_loop(0, B_CHUNK, unroll=4)
        def body(r):
            for k in range(D_MODEL // L):
                obuf[s, r, pl.ds(k * L, L)] = (
                    gbuf[s, r, pl.ds(k * L, L)] * SCALE)

    for s in range(B_NBUF):
        start_gather(s, s)

    n_rounds = (B_UNITS + B_NBUF - 1) // B_NBUF

    def round_body(r, carry):
        for s in range(B_NBUF):
            u = r * B_NBUF + s

            @pl.when(u < B_UNITS)
            def _():
                wait_gather(s)

                @pl.when(u >= B_NBUF)
                def _():
                    wait_out(s)
                scale_unit(s)
                start_out(s, u)

                @pl.when(u + B_NBUF < B_UNITS)
                def _():
                    start_gather(s, u + B_NBUF)
        return carry
    lax.fori_loop(0, n_rounds, round_body, 0)

    for s in range(B_NBUF):
        wait_out(s)


def _build_table():
    mesh = plsc.VectorSubcoreMesh(core_axis_name="c", subcore_axis_name="s")
    return functools.partial(
        pl.kernel,
        mesh=mesh,
        out_type=jax.ShapeDtypeStruct((VOCAB, 128), jnp.float32),
        scratch_types=[
            pltpu.VMEM((A_NBUF, D_MODEL, A_CHUNK), jnp.float32),
            pltpu.VMEM((A_NBUF, A_CHUNK, 128), jnp.float32),
            pltpu.VMEM((A_TAIL, 128), jnp.float32),
        ] + [pltpu.SemaphoreType.DMA] * (2 * A_NBUF),
        compiler_params=pltpu.CompilerParams(
            use_tc_tiling_on_sc=True, needs_layout_passes=False),
    )(_table_kernel)


def _build_lookup():
    mesh = plsc.VectorSubcoreMesh(core_axis_name="c", subcore_axis_name="s")
    return functools.partial(
        pl.kernel,
        mesh=mesh,
        out_type=jax.ShapeDtypeStruct((N_LOOK, D_MODEL), jnp.float32),
        scratch_types=[
            pltpu.VMEM((B_UNITS, B_CHUNK), jnp.int32),
            pltpu.VMEM((B_NBUF, B_CHUNK, 128), jnp.float32),
            pltpu.VMEM((B_NBUF, B_CHUNK, D_MODEL), jnp.float32),
        ] + [pltpu.SemaphoreType.DMA] * (2 * B_NBUF),
        compiler_params=pltpu.CompilerParams(
            use_tc_tiling_on_sc=True, needs_layout_passes=False),
    )(_lookup_kernel)


@jax.jit
def kernel(x, W):
    wt = jnp.swapaxes(W, 0, 1)
    wtail = jnp.pad(W[VOCAB - A_TAIL:, :], ((0, 0), (0, 128 - D_MODEL)))
    tab = _build_table()(wt, wtail)
    idx = x.astype(jnp.int32).reshape(NW, B_UNITS, B_CHUNK)
    out = _build_lookup()(tab, idx)
    return out.reshape(4096, 200, D_MODEL)


# speedup vs baseline: 4.9996x; 1.3369x over previous
import functools
import math

import jax
import jax.numpy as jnp
from jax import lax
from jax.experimental import pallas as pl
from jax.experimental.pallas import tpu as pltpu
from jax.experimental.pallas import tpu_sc as plsc

D_MODEL = 64
SCALE = float(math.sqrt(D_MODEL))
VOCAB = 1000000
NW = 32
L = 16

A_CHUNK = 128
A_UNITS = VOCAB // A_CHUNK
A_TAIL = VOCAB - A_UNITS * A_CHUNK
A_NBUF = 3


def _table_kernel(wt_hbm, wtail_hbm, tab_hbm, ibuf, obuf, tbuf, *sems):
    isems = sems[:A_NBUF]
    osems = sems[A_NBUF:]
    wid = lax.axis_index("s") * 2 + lax.axis_index("c")
    n_units = A_UNITS // NW + jnp.where(wid < A_UNITS % NW, 1, 0)

    def start_in(s, c):
        pltpu.make_async_copy(
            wt_hbm.at[:, pl.ds(c * A_CHUNK, A_CHUNK)], ibuf.at[s],
            isems[s]).start()

    def wait_in(s):
        pltpu.make_async_copy(
            wt_hbm.at[:, pl.ds(0, A_CHUNK)], ibuf.at[s], isems[s]).wait()

    def start_out(s, c):
        pltpu.make_async_copy(
            obuf.at[s], tab_hbm.at[pl.ds(c * A_CHUNK, A_CHUNK)],
            osems[s]).start()

    def wait_out(s):
        pltpu.make_async_copy(
            obuf.at[s], tab_hbm.at[pl.ds(0, A_CHUNK)], osems[s]).wait()

    iota = lax.broadcasted_iota(jnp.int32, (L,), 0)

    def transpose_unit(s):
        @plsc.parallel_loop(0, L, unroll=2)
        def body(j):
            rotv = lax.rem(iota + j, L)
            for k in range(D_MODEL // L):
                dvec = rotv + k * L
                for cb in range(A_CHUNK // L):
                    cvec = iota + cb * L
                    val = plsc.load_gather(ibuf.at[s], [dvec, cvec])
                    plsc.store_scatter(obuf.at[s], [cvec, dvec], val)

    for s in range(A_NBUF):
        start_in(s, s * NW + wid)

    n_rounds = (A_UNITS // NW + 1 + A_NBUF - 1) // A_NBUF

    def round_body(r, carry):
        for s in range(A_NBUF):
            i = r * A_NBUF + s

            @pl.when(i < n_units)
            def _():
                wait_in(s)

                @pl.when(i >= A_NBUF)
                def _():
                    wait_out(s)
                transpose_unit(s)
                start_out(s, i * NW + wid)

                @pl.when(i + A_NBUF < n_units)
                def _():
                    start_in(s, (i + A_NBUF) * NW + wid)
        return carry
    lax.fori_loop(0, n_rounds, round_body, 0)

    for s in range(A_NBUF):
        wait_out(s)

    @pl.when(wid == 0)
    def _tail():
        pltpu.sync_copy(wtail_hbm, tbuf)
        pltpu.sync_copy(tbuf, tab_hbm.at[pl.ds(A_UNITS * A_CHUNK, A_TAIL)])


B_CHUNK = 128
N_LOOK = 4096 * 200
B_UNITS = N_LOOK // (NW * B_CHUNK)
B_NBUF = 3


def _lookup_kernel(tab_hbm, xt_hbm, out_hbm, idx_v, gbuf, obuf, *sems):
    gsems = sems[:B_NBUF]
    osems = sems[B_NBUF:]
    wid = lax.axis_index("s") * 2 + lax.axis_index("c")
    iota = lax.broadcasted_iota(jnp.int32, (L,), 0)
    pltpu.sync_copy(xt_hbm.at[:, pl.ds(wid * B_CHUNK, B_CHUNK)], idx_v)

    def start_gather(s, u):
        pltpu.make_async_copy(
            tab_hbm.at[idx_v.at[u]], gbuf.at[s], gsems[s]).start()

    def wait_gather(s):
        pltpu.make_async_copy(
            tab_hbm.at[idx_v.at[0]], gbuf.at[s], gsems[s]).wait()

    def start_out(s, t):
        pltpu.make_async_copy(
            obuf.at[s], out_hbm.at[t, :, pl.ds(wid * B_CHUNK, B_CHUNK)],
            osems[s]).start()

    def wait_out(s):
        pltpu.make_async_copy(
            obuf.at[s], out_hbm.at[0, :, pl.ds(wid * B_CHUNK, B_CHUNK)],
            osems[s]).wait()

    def scale_unit(s, t):
        colbase = []
        bvecs = []
        for g in range(B_CHUNK // L):
            v = idx_v[t, pl.ds(g * L, L)]
            colbase.append(lax.mul(lax.bitwise_and(v, 1), D_MODEL))
            bvecs.append(iota + g * L)

        @plsc.parallel_loop(0, L, unroll=2)
        def body(j):
            rotv = lax.rem(iota + j, L)
            for k in range(D_MODEL // L):
                dvec = rotv + k * L
                for g in range(B_CHUNK // L):
                    val = plsc.load_gather(
                        gbuf.at[s], [bvecs[g], colbase[g] + dvec])
                    plsc.store_scatter(
                        obuf.at[s], [dvec, bvecs[g]], val * SCALE)

    for s in range(B_NBUF):
        start_gather(s, s)

    n_rounds = (B_UNITS + B_NBUF - 1) // B_NBUF

    def round_body(r, carry):
        for s in range(B_NBUF):
            u = r * B_NBUF + s

            @pl.when(u < B_UNITS)
            def _():
                wait_gather(s)

                @pl.when(u >= B_NBUF)
                def _():
                    wait_out(s)
                scale_unit(s, u)
                start_out(s, u)

                @pl.when(u + B_NBUF < B_UNITS)
                def _():
                    start_gather(s, u + B_NBUF)
        return carry
    lax.fori_loop(0, n_rounds, round_body, 0)

    for s in range(B_NBUF):
        wait_out(s)


def _build_table():
    mesh = plsc.VectorSubcoreMesh(core_axis_name="c", subcore_axis_name="s")
    return functools.partial(
        pl.kernel,
        mesh=mesh,
        out_type=jax.ShapeDtypeStruct((VOCAB, 128), jnp.float32),
        scratch_types=[
            pltpu.VMEM((A_NBUF, D_MODEL, A_CHUNK), jnp.float32),
            pltpu.VMEM((A_NBUF, A_CHUNK, 128), jnp.float32),
            pltpu.VMEM((A_TAIL, 128), jnp.float32),
        ] + [pltpu.SemaphoreType.DMA] * (2 * A_NBUF),
        compiler_params=pltpu.CompilerParams(
            use_tc_tiling_on_sc=True, needs_layout_passes=False),
    )(_table_kernel)


def _build_lookup():
    mesh = plsc.VectorSubcoreMesh(core_axis_name="c", subcore_axis_name="s")
    return functools.partial(
        pl.kernel,
        mesh=mesh,
        out_type=jax.ShapeDtypeStruct((B_UNITS, D_MODEL, 4096), jnp.float32),
        scratch_types=[
            pltpu.VMEM((B_UNITS, B_CHUNK), jnp.int32),
            pltpu.VMEM((B_NBUF, B_CHUNK, 128), jnp.float32),
            pltpu.VMEM((B_NBUF, D_MODEL, B_CHUNK), jnp.float32),
        ] + [pltpu.SemaphoreType.DMA] * (2 * B_NBUF),
        compiler_params=pltpu.CompilerParams(
            use_tc_tiling_on_sc=True, needs_layout_passes=False),
    )(_lookup_kernel)


@jax.jit
def kernel(x, W):
    wt = jnp.swapaxes(W, 0, 1)
    wtail = jnp.pad(W[VOCAB - A_TAIL:, :], ((0, 0), (0, 128 - D_MODEL)))
    tab = _build_table()(wt, wtail)
    xt = jnp.swapaxes(x.astype(jnp.int32), 0, 1)
    out_t = _build_lookup()(tab, xt)
    return jnp.transpose(out_t, (2, 0, 1))
